# baseline (speedup 1.0000x reference)
import jax
import jax.numpy as jnp
from jax.experimental import pallas as pl
from jax.experimental.pallas import tpu as pltpu

_BB = 128


def _fold_fc_kernel(w1_ref, b1_ref, w2_ref, b2_ref, wc_ref, bc_ref):
    full = jnp.dot(w1_ref[...], w2_ref[...],
                   preferred_element_type=jnp.float32)
    wc_ref[...] = jnp.zeros_like(wc_ref)
    for h in range(7):
        wc_ref[256 * h:256 * h + 224, :] = full[224 * h:224 * h + 224, :]
    bc_ref[...] = (jnp.dot(b1_ref[...], w2_ref[...],
                           preferred_element_type=jnp.float32) + b2_ref[...])


def _fwd_kernel(x_ref, a1_ref, b1_ref, a2_ref, b2_ref, wc_ref, bc_ref,
                o_ref, x5_scr, x5b_scr, f_scr):
    BB = x_ref.shape[0]
    f32 = jnp.float32
    xp = x_ref[...]

    xp2 = xp.reshape(BB * 16, 2, 28)
    xe = xp2[:, 0, :].reshape(BB * 8, 2, 28)
    xo = xp2[:, 1, :].reshape(BB * 8, 2, 28)
    ee = xe[:, 0, :].reshape(BB, 8, 28)
    eo = xe[:, 1, :].reshape(BB, 8, 28)
    oe = xo[:, 0, :].reshape(BB, 8, 28)
    oo = xo[:, 1, :].reshape(BB, 8, 28)
    x5_scr[:, 7, 112:168] = jnp.zeros((BB, 56), f32)
    x5_scr[:, :, 0:28] = ee
    x5_scr[:, :, 28:56] = oe
    x5_scr[:, :, 56:84] = eo
    x5_scr[:, :, 84:112] = oo
    x5_scr[:, 0:7, 112:140] = ee[:, 1:8, :]
    x5_scr[:, 0:7, 140:168] = oe[:, 1:8, :]

    z1 = jnp.dot(x5_scr[...].reshape(BB * 8, 168), a1_ref[...],
                 preferred_element_type=f32)
    z1 = jnp.maximum(z1 + b1_ref[...], 0.0)

    p1e = jnp.maximum(jnp.maximum(z1[:, 0:240], z1[:, 256:496]),
                      jnp.maximum(z1[:, 512:752], z1[:, 768:1008]))
    p1o = jnp.maximum(jnp.maximum(z1[:, 1024:1264], z1[:, 1280:1520]),
                      jnp.maximum(z1[:, 1536:1776], z1[:, 1792:2032]))
    p1e = p1e.reshape(BB, 8, 240)
    p1o = p1o.reshape(BB, 8, 240)

    x5b_scr[:, :, 240:256] = jnp.zeros((BB, 8, 16), f32)
    x5b_scr[:, :, 496:512] = jnp.zeros((BB, 8, 16), f32)
    x5b_scr[:, :, 752:768] = jnp.zeros((BB, 8, 16), f32)
    x5b_scr[:, :, 1008:1024] = jnp.zeros((BB, 8, 16), f32)
    x5b_scr[:, 0, 0:240] = jnp.zeros((BB, 240), f32)
    x5b_scr[:, 7, 768:1008] = jnp.zeros((BB, 240), f32)
    x5b_scr[:, 1:8, 0:240] = p1o[:, 0:7, :]
    x5b_scr[:, :, 256:496] = p1e
    x5b_scr[:, :, 512:752] = p1o
    x5b_scr[:, 0:7, 768:1008] = p1e[:, 1:8, :]

    z2 = jnp.dot(x5b_scr[...].reshape(BB * 8, 1024), a2_ref[...],
                 preferred_element_type=f32)
    z2 = jnp.maximum(z2 + b2_ref[...], 0.0)

    pf = jnp.maximum(jnp.maximum(z2[:, 0:224], z2[:, 256:480]),
                     jnp.maximum(z2[:, 512:736], z2[:, 768:992]))
    pf = pf.reshape(BB, 8, 224)

    for h in range(8):
        f_scr[:, 256 * h:256 * h + 224] = pf[:, h, :]
        f_scr[:, 256 * h + 224:256 * h + 256] = jnp.zeros((BB, 32), f32)
    o_ref[...] = (jnp.dot(f_scr[...], wc_ref[...],
                          preferred_element_type=f32) + bc_ref[...])


def kernel(x, A1, bias1, A2, bias2, fc1_w, fc1_b, fc2_w, fc2_b):
    B = x.shape[0]
    BB = _BB if B % _BB == 0 else 1
    f32 = jnp.float32
    xpad = jnp.pad(x.reshape(B, 28, 28), ((0, 0), (2, 2), (0, 0)))

    a1p = jnp.zeros((168, 512 * 4), f32)
    for q in range(4):
        for kh in range(3):
            t = q + kh
            a1p = a1p.at[28 * t:28 * t + 28, 512 * q:512 * q + 240].add(
                A1[kh, 2:30, 0:240])
            a1p = a1p.at[28 * t:28 * t + 28,
                         512 * q + 256:512 * q + 496].add(A1[kh, 2:30,
                                                             240:480])
    b1p = jnp.zeros((1, 2048), f32)
    for q in range(4):
        b1p = b1p.at[:, 512 * q:512 * q + 240].set(bias1[:, 0:240])
        b1p = b1p.at[:, 512 * q + 256:512 * q + 496].set(bias1[:, 240:480])

    a2p = jnp.zeros((1024, 1024), f32)
    for q in range(2):
        for kh in range(3):
            t = q + kh
            a2p = a2p.at[256 * t:256 * t + 240, 512 * q:512 * q + 224].add(
                A2[kh, 16:256, 0:224])
            a2p = a2p.at[256 * t:256 * t + 240,
                         512 * q + 256:512 * q + 480].add(A2[kh, 16:256,
                                                             224:448])
    b2p = jnp.zeros((1, 1024), f32)
    for q in range(2):
        b2p = b2p.at[:, 512 * q:512 * q + 224].set(bias2[:, 0:224])
        b2p = b2p.at[:, 512 * q + 256:512 * q + 480].set(bias2[:, 224:448])

    wc, bc = pl.pallas_call(
        _fold_fc_kernel,
        out_shape=(jax.ShapeDtypeStruct((2048, 8), f32),
                   jax.ShapeDtypeStruct((1, 8), f32)),
    )(fc1_w, fc1_b, fc2_w, fc2_b)

    return pl.pallas_call(
        _fwd_kernel,
        out_shape=jax.ShapeDtypeStruct((B, 8), f32),
        grid=(B // BB,),
        in_specs=[
            pl.BlockSpec((BB, 32, 28), lambda i: (i, 0, 0)),
            pl.BlockSpec((168, 2048), lambda i: (0, 0)),
            pl.BlockSpec((1, 2048), lambda i: (0, 0)),
            pl.BlockSpec((1024, 1024), lambda i: (0, 0)),
            pl.BlockSpec((1, 1024), lambda i: (0, 0)),
            pl.BlockSpec((2048, 8), lambda i: (0, 0)),
            pl.BlockSpec((1, 8), lambda i: (0, 0)),
        ],
        out_specs=pl.BlockSpec((BB, 8), lambda i: (i, 0)),
        scratch_shapes=[
            pltpu.VMEM((BB, 8, 168), f32),
            pltpu.VMEM((BB, 8, 1024), f32),
            pltpu.VMEM((BB, 2048), f32),
        ],
        compiler_params=pltpu.CompilerParams(
            dimension_semantics=("parallel",)),
    )(xpad, a1p, b1p, a2p, b2p, wc, bc)


# speedup vs baseline: 1.3246x; 1.3246x over previous
import jax
import jax.numpy as jnp
from jax.experimental import pallas as pl
from jax.experimental.pallas import tpu as pltpu

_BB = 128


def _prep_kernel(a1_ref, b1_ref, a2_ref, b2_ref, w1_ref, b1f_ref, w2_ref,
                 b2f_ref, a1p_ref, b1p_ref, a2p_ref, b2p_ref, wc_ref, bc_ref):
    a1p_ref[...] = jnp.zeros_like(a1p_ref)
    b1p_ref[...] = jnp.zeros_like(b1p_ref)
    for q in range(4):
        for kh in range(3):
            t = q + kh
            a1p_ref[28 * t:28 * t + 28, 512 * q:512 * q + 240] = \
                a1_ref[kh, 2:30, 0:240]
            a1p_ref[28 * t:28 * t + 28, 512 * q + 256:512 * q + 496] = \
                a1_ref[kh, 2:30, 240:480]
        b1p_ref[:, 512 * q:512 * q + 240] = b1_ref[:, 0:240]
        b1p_ref[:, 512 * q + 256:512 * q + 496] = b1_ref[:, 240:480]

    a2p_ref[...] = jnp.zeros_like(a2p_ref)
    b2p_ref[...] = jnp.zeros_like(b2p_ref)
    for q in range(2):
        for kh in range(3):
            t = q + kh
            a2p_ref[256 * t:256 * t + 240, 512 * q:512 * q + 224] = \
                a2_ref[kh, 16:256, 0:224]
            a2p_ref[256 * t:256 * t + 240, 512 * q + 256:512 * q + 480] = \
                a2_ref[kh, 16:256, 224:448]
        b2p_ref[:, 512 * q:512 * q + 224] = b2_ref[:, 0:224]
        b2p_ref[:, 512 * q + 256:512 * q + 480] = b2_ref[:, 224:448]

    full = jnp.dot(w1_ref[...], w2_ref[...],
                   preferred_element_type=jnp.float32)
    wc_ref[...] = jnp.zeros_like(wc_ref)
    for h in range(7):
        wc_ref[256 * h:256 * h + 224, :] = full[224 * h:224 * h + 224, :]
    bc_ref[...] = (jnp.dot(b1f_ref[...], w2_ref[...],
                           preferred_element_type=jnp.float32) + b2f_ref[...])


def _fwd_kernel(x_ref, a1_ref, b1_ref, a2_ref, b2_ref, wc_ref, bc_ref,
                o_ref, x5_scr, x5b_scr, f_scr):
    BB = x_ref.shape[0]
    f32 = jnp.float32
    xp = x_ref[...]

    xp2 = xp.reshape(BB * 16, 2, 28)
    xe = xp2[:, 0, :].reshape(BB * 8, 2, 28)
    xo = xp2[:, 1, :].reshape(BB * 8, 2, 28)
    ee = xe[:, 0, :].reshape(BB, 8, 28)
    eo = xe[:, 1, :].reshape(BB, 8, 28)
    oe = xo[:, 0, :].reshape(BB, 8, 28)
    oo = xo[:, 1, :].reshape(BB, 8, 28)
    x5_scr[:, 7, 112:168] = jnp.zeros((BB, 56), f32)
    x5_scr[:, :, 0:28] = ee
    x5_scr[:, :, 28:56] = oe
    x5_scr[:, :, 56:84] = eo
    x5_scr[:, :, 84:112] = oo
    x5_scr[:, 0:7, 112:140] = ee[:, 1:8, :]
    x5_scr[:, 0:7, 140:168] = oe[:, 1:8, :]

    z1 = jnp.dot(x5_scr[...].reshape(BB * 8, 168), a1_ref[...],
                 preferred_element_type=f32)
    z1 = jnp.maximum(z1 + b1_ref[...], 0.0)

    p1e = jnp.maximum(jnp.maximum(z1[:, 0:240], z1[:, 256:496]),
                      jnp.maximum(z1[:, 512:752], z1[:, 768:1008]))
    p1o = jnp.maximum(jnp.maximum(z1[:, 1024:1264], z1[:, 1280:1520]),
                      jnp.maximum(z1[:, 1536:1776], z1[:, 1792:2032]))
    p1e = p1e.reshape(BB, 8, 240)
    p1o = p1o.reshape(BB, 8, 240)

    x5b_scr[:, :, 240:256] = jnp.zeros((BB, 8, 16), f32)
    x5b_scr[:, :, 496:512] = jnp.zeros((BB, 8, 16), f32)
    x5b_scr[:, :, 752:768] = jnp.zeros((BB, 8, 16), f32)
    x5b_scr[:, :, 1008:1024] = jnp.zeros((BB, 8, 16), f32)
    x5b_scr[:, 0, 0:240] = jnp.zeros((BB, 240), f32)
    x5b_scr[:, 7, 768:1008] = jnp.zeros((BB, 240), f32)
    x5b_scr[:, 1:8, 0:240] = p1o[:, 0:7, :]
    x5b_scr[:, :, 256:496] = p1e
    x5b_scr[:, :, 512:752] = p1o
    x5b_scr[:, 0:7, 768:1008] = p1e[:, 1:8, :]

    z2 = jnp.dot(x5b_scr[...].reshape(BB * 8, 1024), a2_ref[...],
                 preferred_element_type=f32)
    z2 = jnp.maximum(z2 + b2_ref[...], 0.0)

    pf = jnp.maximum(jnp.maximum(z2[:, 0:224], z2[:, 256:480]),
                     jnp.maximum(z2[:, 512:736], z2[:, 768:992]))
    pf = pf.reshape(BB, 8, 224)

    for h in range(8):
        f_scr[:, 256 * h:256 * h + 224] = pf[:, h, :]
        f_scr[:, 256 * h + 224:256 * h + 256] = jnp.zeros((BB, 32), f32)
    o_ref[...] = (jnp.dot(f_scr[...], wc_ref[...],
                          preferred_element_type=f32) + bc_ref[...])


def kernel(x, A1, bias1, A2, bias2, fc1_w, fc1_b, fc2_w, fc2_b):
    B = x.shape[0]
    BB = _BB if B % _BB == 0 else 1
    f32 = jnp.float32
    xpad = jnp.pad(x.reshape(B, 28, 28), ((0, 0), (2, 2), (0, 0)))

    a1p, b1p, a2p, b2p, wc, bc = pl.pallas_call(
        _prep_kernel,
        out_shape=(jax.ShapeDtypeStruct((168, 2048), f32),
                   jax.ShapeDtypeStruct((1, 2048), f32),
                   jax.ShapeDtypeStruct((1024, 1024), f32),
                   jax.ShapeDtypeStruct((1, 1024), f32),
                   jax.ShapeDtypeStruct((2048, 8), f32),
                   jax.ShapeDtypeStruct((1, 8), f32)),
    )(A1, bias1, A2, bias2, fc1_w, fc1_b, fc2_w, fc2_b)

    return pl.pallas_call(
        _fwd_kernel,
        out_shape=jax.ShapeDtypeStruct((B, 8), f32),
        grid=(B // BB,),
        in_specs=[
            pl.BlockSpec((BB, 32, 28), lambda i: (i, 0, 0)),
            pl.BlockSpec((168, 2048), lambda i: (0, 0)),
            pl.BlockSpec((1, 2048), lambda i: (0, 0)),
            pl.BlockSpec((1024, 1024), lambda i: (0, 0)),
            pl.BlockSpec((1, 1024), lambda i: (0, 0)),
            pl.BlockSpec((2048, 8), lambda i: (0, 0)),
            pl.BlockSpec((1, 8), lambda i: (0, 0)),
        ],
        out_specs=pl.BlockSpec((BB, 8), lambda i: (i, 0)),
        scratch_shapes=[
            pltpu.VMEM((BB, 8, 168), f32),
            pltpu.VMEM((BB, 8, 1024), f32),
            pltpu.VMEM((BB, 2048), f32),
        ],
        compiler_params=pltpu.CompilerParams(
            dimension_semantics=("parallel",)),
    )(xpad, a1p, b1p, a2p, b2p, wc, bc)


# speedup vs baseline: 1.3493x; 1.0187x over previous
import jax
import jax.numpy as jnp
from jax.experimental import pallas as pl
from jax.experimental.pallas import tpu as pltpu

_BB = 128


def _prep_kernel(a1_ref, b1_ref, a2_ref, b2_ref, w1_ref, b1f_ref, w2_ref,
                 b2f_ref, a1p_ref, b1p_ref, a2p_ref, b2p_ref, wc_ref, bc_ref):
    a1p_ref[...] = jnp.zeros_like(a1p_ref)
    b1p_ref[...] = jnp.zeros_like(b1p_ref)
    for q in range(4):
        for kh in range(3):
            t = q + kh
            a1p_ref[28 * t:28 * t + 28, 512 * q:512 * q + 240] = \
                a1_ref[kh, 2:30, 0:240].astype(jnp.bfloat16)
            a1p_ref[28 * t:28 * t + 28, 512 * q + 256:512 * q + 496] = \
                a1_ref[kh, 2:30, 240:480].astype(jnp.bfloat16)
        b1p_ref[:, 512 * q:512 * q + 240] = b1_ref[:, 0:240]
        b1p_ref[:, 512 * q + 256:512 * q + 496] = b1_ref[:, 240:480]

    a2p_ref[...] = jnp.zeros_like(a2p_ref)
    b2p_ref[...] = jnp.zeros_like(b2p_ref)
    for q in range(2):
        for kh in range(3):
            t = q + kh
            a2p_ref[256 * t:256 * t + 240, 512 * q:512 * q + 224] = \
                a2_ref[kh, 16:256, 0:224].astype(jnp.bfloat16)
            a2p_ref[256 * t:256 * t + 240, 512 * q + 256:512 * q + 480] = \
                a2_ref[kh, 16:256, 224:448].astype(jnp.bfloat16)
        b2p_ref[:, 512 * q:512 * q + 224] = b2_ref[:, 0:224]
        b2p_ref[:, 512 * q + 256:512 * q + 480] = b2_ref[:, 224:448]

    full = jnp.dot(w1_ref[...], w2_ref[...],
                   preferred_element_type=jnp.float32)
    wc_ref[...] = jnp.zeros_like(wc_ref)
    for h in range(7):
        wc_ref[256 * h:256 * h + 224, :] = \
            full[224 * h:224 * h + 224, :].astype(jnp.bfloat16)
    bc_ref[...] = (jnp.dot(b1f_ref[...], w2_ref[...],
                           preferred_element_type=jnp.float32) + b2f_ref[...])


def _fwd_kernel(x_ref, a1_ref, b1_ref, a2_ref, b2_ref, wc_ref, bc_ref,
                o_ref, x5_scr, x5b_scr, f_scr):
    BB = x_ref.shape[0]
    f32 = jnp.float32
    xp = x_ref[...]

    xp2 = xp.reshape(BB * 16, 2, 28)
    xe = xp2[:, 0, :].reshape(BB * 8, 2, 28)
    xo = xp2[:, 1, :].reshape(BB * 8, 2, 28)
    ee = xe[:, 0, :].reshape(BB, 8, 28)
    eo = xe[:, 1, :].reshape(BB, 8, 28)
    oe = xo[:, 0, :].reshape(BB, 8, 28)
    oo = xo[:, 1, :].reshape(BB, 8, 28)
    x5_scr[:, 7, 112:168] = jnp.zeros((BB, 56), f32)
    x5_scr[:, :, 0:28] = ee
    x5_scr[:, :, 28:56] = oe
    x5_scr[:, :, 56:84] = eo
    x5_scr[:, :, 84:112] = oo
    x5_scr[:, 0:7, 112:140] = ee[:, 1:8, :]
    x5_scr[:, 0:7, 140:168] = oe[:, 1:8, :]

    z1 = jnp.dot(x5_scr[...].reshape(BB * 8, 168).astype(jnp.bfloat16),
                 a1_ref[...],
                 preferred_element_type=f32)
    z1 = jnp.maximum(z1 + b1_ref[...], 0.0)

    p1e = jnp.maximum(jnp.maximum(z1[:, 0:240], z1[:, 256:496]),
                      jnp.maximum(z1[:, 512:752], z1[:, 768:1008]))
    p1o = jnp.maximum(jnp.maximum(z1[:, 1024:1264], z1[:, 1280:1520]),
                      jnp.maximum(z1[:, 1536:1776], z1[:, 1792:2032]))
    p1e = p1e.reshape(BB, 8, 240)
    p1o = p1o.reshape(BB, 8, 240)

    x5b_scr[:, :, 240:256] = jnp.zeros((BB, 8, 16), f32)
    x5b_scr[:, :, 496:512] = jnp.zeros((BB, 8, 16), f32)
    x5b_scr[:, :, 752:768] = jnp.zeros((BB, 8, 16), f32)
    x5b_scr[:, :, 1008:1024] = jnp.zeros((BB, 8, 16), f32)
    x5b_scr[:, 0, 0:240] = jnp.zeros((BB, 240), f32)
    x5b_scr[:, 7, 768:1008] = jnp.zeros((BB, 240), f32)
    x5b_scr[:, 1:8, 0:240] = p1o[:, 0:7, :]
    x5b_scr[:, :, 256:496] = p1e
    x5b_scr[:, :, 512:752] = p1o
    x5b_scr[:, 0:7, 768:1008] = p1e[:, 1:8, :]

    z2 = jnp.dot(x5b_scr[...].reshape(BB * 8, 1024).astype(jnp.bfloat16),
                 a2_ref[...],
                 preferred_element_type=f32)
    z2 = jnp.maximum(z2 + b2_ref[...], 0.0)

    pf = jnp.maximum(jnp.maximum(z2[:, 0:224], z2[:, 256:480]),
                     jnp.maximum(z2[:, 512:736], z2[:, 768:992]))
    pf = pf.reshape(BB, 8, 224)

    for h in range(8):
        f_scr[:, 256 * h:256 * h + 224] = pf[:, h, :]
        f_scr[:, 256 * h + 224:256 * h + 256] = jnp.zeros((BB, 32), f32)
    o_ref[...] = (jnp.dot(f_scr[...].astype(jnp.bfloat16), wc_ref[...],
                          preferred_element_type=f32) + bc_ref[...])


def kernel(x, A1, bias1, A2, bias2, fc1_w, fc1_b, fc2_w, fc2_b):
    B = x.shape[0]
    BB = _BB if B % _BB == 0 else 1
    f32 = jnp.float32
    xpad = jnp.pad(x.reshape(B, 28, 28), ((0, 0), (2, 2), (0, 0)))

    a1p, b1p, a2p, b2p, wc, bc = pl.pallas_call(
        _prep_kernel,
        out_shape=(jax.ShapeDtypeStruct((168, 2048), jnp.bfloat16),
                   jax.ShapeDtypeStruct((1, 2048), f32),
                   jax.ShapeDtypeStruct((1024, 1024), jnp.bfloat16),
                   jax.ShapeDtypeStruct((1, 1024), f32),
                   jax.ShapeDtypeStruct((2048, 8), jnp.bfloat16),
                   jax.ShapeDtypeStruct((1, 8), f32)),
    )(A1, bias1, A2, bias2, fc1_w, fc1_b, fc2_w, fc2_b)

    return pl.pallas_call(
        _fwd_kernel,
        out_shape=jax.ShapeDtypeStruct((B, 8), f32),
        grid=(B // BB,),
        in_specs=[
            pl.BlockSpec((BB, 32, 28), lambda i: (i, 0, 0)),
            pl.BlockSpec((168, 2048), lambda i: (0, 0)),
            pl.BlockSpec((1, 2048), lambda i: (0, 0)),
            pl.BlockSpec((1024, 1024), lambda i: (0, 0)),
            pl.BlockSpec((1, 1024), lambda i: (0, 0)),
            pl.BlockSpec((2048, 8), lambda i: (0, 0)),
            pl.BlockSpec((1, 8), lambda i: (0, 0)),
        ],
        out_specs=pl.BlockSpec((BB, 8), lambda i: (i, 0)),
        scratch_shapes=[
            pltpu.VMEM((BB, 8, 168), f32),
            pltpu.VMEM((BB, 8, 1024), f32),
            pltpu.VMEM((BB, 2048), f32),
        ],
        compiler_params=pltpu.CompilerParams(
            dimension_semantics=("parallel",)),
    )(xpad, a1p, b1p, a2p, b2p, wc, bc)


# speedup vs baseline: 1.6991x; 1.2592x over previous
import jax
import jax.numpy as jnp
from jax.experimental import pallas as pl
from jax.experimental.pallas import tpu as pltpu

_BB = 128


def _prep_kernel(a1_ref, b1_ref, a2_ref, b2_ref, w1_ref, b1f_ref, w2_ref,
                 b2f_ref, a1p_ref, b1p_ref, a2p_ref, b2p_ref, wc_ref, bc_ref):
    a1p_ref[...] = jnp.zeros_like(a1p_ref)
    b1p_ref[...] = jnp.zeros_like(b1p_ref)
    for q in range(4):
        for kh in range(3):
            t = q + kh
            a1p_ref[28 * t:28 * t + 28, 512 * q:512 * q + 240] = \
                a1_ref[kh, 2:30, 0:240].astype(jnp.bfloat16)
            a1p_ref[28 * t:28 * t + 28, 512 * q + 256:512 * q + 496] = \
                a1_ref[kh, 2:30, 240:480].astype(jnp.bfloat16)
        b1p_ref[:, 512 * q:512 * q + 240] = b1_ref[:, 0:240]
        b1p_ref[:, 512 * q + 256:512 * q + 496] = b1_ref[:, 240:480]

    a2p_ref[...] = jnp.zeros_like(a2p_ref)
    b2p_ref[...] = jnp.zeros_like(b2p_ref)
    for q in range(2):
        for kh in range(3):
            t = q + kh
            a2p_ref[256 * t:256 * t + 240, 512 * q:512 * q + 224] = \
                a2_ref[kh, 16:256, 0:224].astype(jnp.bfloat16)
            a2p_ref[256 * t:256 * t + 240, 512 * q + 256:512 * q + 480] = \
                a2_ref[kh, 16:256, 224:448].astype(jnp.bfloat16)
        b2p_ref[:, 512 * q:512 * q + 224] = b2_ref[:, 0:224]
        b2p_ref[:, 512 * q + 256:512 * q + 480] = b2_ref[:, 224:448]

    full = jnp.dot(w1_ref[...], w2_ref[...],
                   preferred_element_type=jnp.float32)
    wc_ref[...] = jnp.zeros_like(wc_ref)
    for h in range(7):
        wc_ref[256 * h:256 * h + 224, :] = \
            full[224 * h:224 * h + 224, :].astype(jnp.bfloat16)
    bc_ref[...] = (jnp.dot(b1f_ref[...], w2_ref[...],
                           preferred_element_type=jnp.float32) + b2f_ref[...])


def _fwd_kernel(x_ref, a1_ref, b1_ref, a2_ref, b2_ref, wc_ref, bc_ref,
                o_ref, x5_scr, x5b_scr, f_scr):
    BB = x_ref.shape[0]
    f32 = jnp.float32
    xp = x_ref[...]

    xm = xp.reshape(BB * 7, 4, 28)
    xm0 = xm[:, 0, :].reshape(BB, 7, 28)
    xm1 = xm[:, 1, :].reshape(BB, 7, 28)
    xm2 = xm[:, 2, :].reshape(BB, 7, 28)
    xm3 = xm[:, 3, :].reshape(BB, 7, 28)
    x5_scr[:, 0, 0:56] = jnp.zeros((BB, 56), f32)
    x5_scr[:, 7, 56:168] = jnp.zeros((BB, 112), f32)
    x5_scr[:, 1:8, 0:28] = xm2
    x5_scr[:, 1:8, 28:56] = xm3
    x5_scr[:, 0:7, 56:84] = xm0
    x5_scr[:, 0:7, 84:112] = xm1
    x5_scr[:, 0:7, 112:140] = xm2
    x5_scr[:, 0:7, 140:168] = xm3

    z1 = jnp.dot(x5_scr[...].reshape(BB * 8, 168).astype(jnp.bfloat16),
                 a1_ref[...],
                 preferred_element_type=f32)
    z1 = jnp.maximum(z1 + b1_ref[...], 0.0)

    p1e = jnp.maximum(jnp.maximum(z1[:, 0:240], z1[:, 256:496]),
                      jnp.maximum(z1[:, 512:752], z1[:, 768:1008]))
    p1o = jnp.maximum(jnp.maximum(z1[:, 1024:1264], z1[:, 1280:1520]),
                      jnp.maximum(z1[:, 1536:1776], z1[:, 1792:2032]))
    p1e = p1e.reshape(BB, 8, 240)
    p1o = p1o.reshape(BB, 8, 240)

    x5b_scr[:, :, 240:256] = jnp.zeros((BB, 8, 16), f32)
    x5b_scr[:, :, 496:512] = jnp.zeros((BB, 8, 16), f32)
    x5b_scr[:, :, 752:768] = jnp.zeros((BB, 8, 16), f32)
    x5b_scr[:, :, 1008:1024] = jnp.zeros((BB, 8, 16), f32)
    x5b_scr[:, 0, 0:240] = jnp.zeros((BB, 240), f32)
    x5b_scr[:, 7, 768:1008] = jnp.zeros((BB, 240), f32)
    x5b_scr[:, 1:8, 0:240] = p1o[:, 0:7, :]
    x5b_scr[:, :, 256:496] = p1e
    x5b_scr[:, :, 512:752] = p1o
    x5b_scr[:, 0:7, 768:1008] = p1e[:, 1:8, :]

    z2 = jnp.dot(x5b_scr[...].reshape(BB * 8, 1024).astype(jnp.bfloat16),
                 a2_ref[...],
                 preferred_element_type=f32)
    z2 = jnp.maximum(z2 + b2_ref[...], 0.0)

    pf = jnp.maximum(jnp.maximum(z2[:, 0:224], z2[:, 256:480]),
                     jnp.maximum(z2[:, 512:736], z2[:, 768:992]))
    pf = pf.reshape(BB, 8, 224)

    for h in range(8):
        f_scr[:, 256 * h:256 * h + 224] = pf[:, h, :]
        f_scr[:, 256 * h + 224:256 * h + 256] = jnp.zeros((BB, 32), f32)
    o_ref[...] = (jnp.dot(f_scr[...].astype(jnp.bfloat16), wc_ref[...],
                          preferred_element_type=f32) + bc_ref[...])


def kernel(x, A1, bias1, A2, bias2, fc1_w, fc1_b, fc2_w, fc2_b):
    B = x.shape[0]
    BB = _BB if B % _BB == 0 else 1
    f32 = jnp.float32
    xs = x.reshape(B, 28, 28)

    a1p, b1p, a2p, b2p, wc, bc = pl.pallas_call(
        _prep_kernel,
        out_shape=(jax.ShapeDtypeStruct((168, 2048), jnp.bfloat16),
                   jax.ShapeDtypeStruct((1, 2048), f32),
                   jax.ShapeDtypeStruct((1024, 1024), jnp.bfloat16),
                   jax.ShapeDtypeStruct((1, 1024), f32),
                   jax.ShapeDtypeStruct((2048, 8), jnp.bfloat16),
                   jax.ShapeDtypeStruct((1, 8), f32)),
    )(A1, bias1, A2, bias2, fc1_w, fc1_b, fc2_w, fc2_b)

    return pl.pallas_call(
        _fwd_kernel,
        out_shape=jax.ShapeDtypeStruct((B, 8), f32),
        grid=(B // BB,),
        in_specs=[
            pl.BlockSpec((BB, 28, 28), lambda i: (i, 0, 0)),
            pl.BlockSpec((168, 2048), lambda i: (0, 0)),
            pl.BlockSpec((1, 2048), lambda i: (0, 0)),
            pl.BlockSpec((1024, 1024), lambda i: (0, 0)),
            pl.BlockSpec((1, 1024), lambda i: (0, 0)),
            pl.BlockSpec((2048, 8), lambda i: (0, 0)),
            pl.BlockSpec((1, 8), lambda i: (0, 0)),
        ],
        out_specs=pl.BlockSpec((BB, 8), lambda i: (i, 0)),
        scratch_shapes=[
            pltpu.VMEM((BB, 8, 168), f32),
            pltpu.VMEM((BB, 8, 1024), f32),
            pltpu.VMEM((BB, 2048), f32),
        ],
        compiler_params=pltpu.CompilerParams(
            dimension_semantics=("parallel",)),
    )(xs, a1p, b1p, a2p, b2p, wc, bc)


# speedup vs baseline: 1.7391x; 1.0235x over previous
import jax
import jax.numpy as jnp
from jax.experimental import pallas as pl
from jax.experimental.pallas import tpu as pltpu

_BB = 256


def _prep_kernel(a1_ref, b1_ref, a2_ref, b2_ref, w1_ref, b1f_ref, w2_ref,
                 b2f_ref, a1p_ref, b1p_ref, a2p_ref, b2p_ref, wc_ref, bc_ref):
    a1p_ref[...] = jnp.zeros_like(a1p_ref)
    b1p_ref[...] = jnp.zeros_like(b1p_ref)
    for q in range(4):
        for kh in range(3):
            t = q + kh
            a1p_ref[28 * t:28 * t + 28, 512 * q:512 * q + 240] = \
                a1_ref[kh, 2:30, 0:240].astype(jnp.bfloat16)
            a1p_ref[28 * t:28 * t + 28, 512 * q + 256:512 * q + 496] = \
                a1_ref[kh, 2:30, 240:480].astype(jnp.bfloat16)
        b1p_ref[:, 512 * q:512 * q + 240] = b1_ref[:, 0:240]
        b1p_ref[:, 512 * q + 256:512 * q + 496] = b1_ref[:, 240:480]

    a2p_ref[...] = jnp.zeros_like(a2p_ref)
    b2p_ref[...] = jnp.zeros_like(b2p_ref)
    for q in range(2):
        for kh in range(3):
            t = q + kh
            a2p_ref[256 * t:256 * t + 240, 512 * q:512 * q + 224] = \
                a2_ref[kh, 16:256, 0:224].astype(jnp.bfloat16)
            a2p_ref[256 * t:256 * t + 240, 512 * q + 256:512 * q + 480] = \
                a2_ref[kh, 16:256, 224:448].astype(jnp.bfloat16)
        b2p_ref[:, 512 * q:512 * q + 224] = b2_ref[:, 0:224]
        b2p_ref[:, 512 * q + 256:512 * q + 480] = b2_ref[:, 224:448]

    full = jnp.dot(w1_ref[...], w2_ref[...],
                   preferred_element_type=jnp.float32)
    wc_ref[...] = jnp.zeros_like(wc_ref)
    for h in range(7):
        wc_ref[256 * h:256 * h + 224, :] = \
            full[224 * h:224 * h + 224, :].astype(jnp.bfloat16)
    bc_ref[...] = (jnp.dot(b1f_ref[...], w2_ref[...],
                           preferred_element_type=jnp.float32) + b2f_ref[...])


def _fwd_kernel(x_ref, a1_ref, b1_ref, a2_ref, b2_ref, wc_ref, bc_ref,
                o_ref, x5_scr, x5b_scr, f_scr):
    BB = x_ref.shape[0]
    f32 = jnp.float32
    xp = x_ref[...]

    xm = xp.reshape(BB * 7, 4, 28)
    xm0 = xm[:, 0, :].reshape(BB, 7, 28)
    xm1 = xm[:, 1, :].reshape(BB, 7, 28)
    xm2 = xm[:, 2, :].reshape(BB, 7, 28)
    xm3 = xm[:, 3, :].reshape(BB, 7, 28)
    x5_scr[:, 0, 0:56] = jnp.zeros((BB, 56), f32)
    x5_scr[:, 7, 56:168] = jnp.zeros((BB, 112), f32)
    x5_scr[:, 1:8, 0:28] = xm2
    x5_scr[:, 1:8, 28:56] = xm3
    x5_scr[:, 0:7, 56:84] = xm0
    x5_scr[:, 0:7, 84:112] = xm1
    x5_scr[:, 0:7, 112:140] = xm2
    x5_scr[:, 0:7, 140:168] = xm3

    z1 = jnp.dot(x5_scr[...].reshape(BB * 8, 168).astype(jnp.bfloat16),
                 a1_ref[...],
                 preferred_element_type=f32)
    z1 = jnp.maximum(z1 + b1_ref[...], 0.0)

    p1e = jnp.maximum(jnp.maximum(z1[:, 0:240], z1[:, 256:496]),
                      jnp.maximum(z1[:, 512:752], z1[:, 768:1008]))
    p1o = jnp.maximum(jnp.maximum(z1[:, 1024:1264], z1[:, 1280:1520]),
                      jnp.maximum(z1[:, 1536:1776], z1[:, 1792:2032]))
    p1e = p1e.reshape(BB, 8, 240)
    p1o = p1o.reshape(BB, 8, 240)

    x5b_scr[:, :, 240:256] = jnp.zeros((BB, 8, 16), f32)
    x5b_scr[:, :, 496:512] = jnp.zeros((BB, 8, 16), f32)
    x5b_scr[:, :, 752:768] = jnp.zeros((BB, 8, 16), f32)
    x5b_scr[:, :, 1008:1024] = jnp.zeros((BB, 8, 16), f32)
    x5b_scr[:, 0, 0:240] = jnp.zeros((BB, 240), f32)
    x5b_scr[:, 7, 768:1008] = jnp.zeros((BB, 240), f32)
    x5b_scr[:, 1:8, 0:240] = p1o[:, 0:7, :]
    x5b_scr[:, :, 256:496] = p1e
    x5b_scr[:, :, 512:752] = p1o
    x5b_scr[:, 0:7, 768:1008] = p1e[:, 1:8, :]

    z2 = jnp.dot(x5b_scr[...].reshape(BB * 8, 1024).astype(jnp.bfloat16),
                 a2_ref[...],
                 preferred_element_type=f32)
    z2 = jnp.maximum(z2 + b2_ref[...], 0.0)

    pf = jnp.maximum(jnp.maximum(z2[:, 0:224], z2[:, 256:480]),
                     jnp.maximum(z2[:, 512:736], z2[:, 768:992]))
    pf = pf.reshape(BB, 8, 224)

    for h in range(8):
        f_scr[:, 256 * h:256 * h + 224] = pf[:, h, :]
        f_scr[:, 256 * h + 224:256 * h + 256] = jnp.zeros((BB, 32), f32)
    o_ref[...] = (jnp.dot(f_scr[...].astype(jnp.bfloat16), wc_ref[...],
                          preferred_element_type=f32) + bc_ref[...])


def kernel(x, A1, bias1, A2, bias2, fc1_w, fc1_b, fc2_w, fc2_b):
    B = x.shape[0]
    BB = _BB if B % _BB == 0 else 1
    f32 = jnp.float32
    xs = x.reshape(B, 28, 28)

    a1p, b1p, a2p, b2p, wc, bc = pl.pallas_call(
        _prep_kernel,
        out_shape=(jax.ShapeDtypeStruct((168, 2048), jnp.bfloat16),
                   jax.ShapeDtypeStruct((1, 2048), f32),
                   jax.ShapeDtypeStruct((1024, 1024), jnp.bfloat16),
                   jax.ShapeDtypeStruct((1, 1024), f32),
                   jax.ShapeDtypeStruct((2048, 8), jnp.bfloat16),
                   jax.ShapeDtypeStruct((1, 8), f32)),
    )(A1, bias1, A2, bias2, fc1_w, fc1_b, fc2_w, fc2_b)

    return pl.pallas_call(
        _fwd_kernel,
        out_shape=jax.ShapeDtypeStruct((B, 8), f32),
        grid=(B // BB,),
        in_specs=[
            pl.BlockSpec((BB, 28, 28), lambda i: (i, 0, 0)),
            pl.BlockSpec((168, 2048), lambda i: (0, 0)),
            pl.BlockSpec((1, 2048), lambda i: (0, 0)),
            pl.BlockSpec((1024, 1024), lambda i: (0, 0)),
            pl.BlockSpec((1, 1024), lambda i: (0, 0)),
            pl.BlockSpec((2048, 8), lambda i: (0, 0)),
            pl.BlockSpec((1, 8), lambda i: (0, 0)),
        ],
        out_specs=pl.BlockSpec((BB, 8), lambda i: (i, 0)),
        scratch_shapes=[
            pltpu.VMEM((BB, 8, 168), f32),
            pltpu.VMEM((BB, 8, 1024), f32),
            pltpu.VMEM((BB, 2048), f32),
        ],
        compiler_params=pltpu.CompilerParams(
            dimension_semantics=("parallel",)),
    )(xs, a1p, b1p, a2p, b2p, wc, bc)
